# Initial kernel scaffold; baseline (speedup 1.0000x reference)
import jax
import jax.numpy as jnp
from jax.experimental import pallas as pl


def kernel(x, Wg, bg, We, be):
    raise NotImplementedError("write your pallas kernel here")


# speedup vs baseline: 1.0022x; 1.0022x over previous
import functools

import jax
import jax.numpy as jnp
from jax.experimental import pallas as pl
from jax.experimental.pallas import tpu as pltpu

S = 2048
D = 2048
E = 8
TO = 256


def _moe_body(x_ref, wgt_ref, bg_ref, we_ref, be_ref, out_ref, g_s, xb_s):
    o = pl.program_id(0)
    e = pl.program_id(1)

    @pl.when((o == 0) & (e == 0))
    def _router():
        xb_s[...] = x_ref[...].astype(jnp.bfloat16)
        logits = jax.lax.dot_general(
            x_ref[...], wgt_ref[...],
            (((1,), (0,)), ((), ())),
            preferred_element_type=jnp.float32,
        ) + bg_ref[...]
        lane = jax.lax.broadcasted_iota(jnp.int32, (S, E), 1)
        m1 = jnp.max(logits, axis=1, keepdims=True)
        i1 = jnp.min(jnp.where(logits == m1, lane, E), axis=1, keepdims=True)
        oh1 = lane == i1
        masked = jnp.where(oh1, -jnp.inf, logits)
        m2 = jnp.max(masked, axis=1, keepdims=True)
        i2 = jnp.min(jnp.where(masked == m2, lane, E), axis=1, keepdims=True)
        oh2 = lane == i2
        z = jnp.exp(m2 - m1)
        w1 = 1.0 / (1.0 + z)
        w2 = 1.0 - w1
        g_s[...] = jnp.where(oh1, w1, 0.0) + jnp.where(oh2, w2, 0.0)

    lane = jax.lax.broadcasted_iota(jnp.int32, (S, E), 1)
    g = g_s[...]
    gcol = jnp.sum(jnp.where(lane == e, g, 0.0), axis=1, keepdims=True)

    wb = we_ref[0].astype(jnp.bfloat16)
    prod = jax.lax.dot_general(
        xb_s[...], wb,
        (((1,), (1,)), ((), ())),
        preferred_element_type=jnp.float32,
    )
    contrib = gcol * prod + gcol * be_ref[0]

    @pl.when(e == 0)
    def _init():
        out_ref[...] = contrib

    @pl.when(e != 0)
    def _acc():
        out_ref[...] += contrib


@functools.partial(jax.jit, static_argnames=())
def _moe(x2d, wgt, bg2d, we, be):
    n_o = D // TO
    return pl.pallas_call(
        _moe_body,
        grid=(n_o, E),
        in_specs=[
            pl.BlockSpec((S, D), lambda o, e: (0, 0)),
            pl.BlockSpec((D, E), lambda o, e: (0, 0)),
            pl.BlockSpec((1, E), lambda o, e: (0, 0)),
            pl.BlockSpec((1, TO, D), lambda o, e: (e, o, 0)),
            pl.BlockSpec((1, 1, TO), lambda o, e: (e, 0, o)),
        ],
        out_specs=pl.BlockSpec((S, TO), lambda o, e: (0, o)),
        out_shape=jax.ShapeDtypeStruct((S, D), jnp.float32),
        scratch_shapes=[
            pltpu.VMEM((S, E), jnp.float32),
            pltpu.VMEM((S, D), jnp.bfloat16),
        ],
    )(x2d, wgt, bg2d, we, be)


def kernel(x, Wg, bg, We, be):
    B, S_, D_ = x.shape
    x2d = x.reshape(S_, D_)
    out = _moe(x2d, Wg.T, bg.reshape(1, E), We, be.reshape(E, 1, D))
    return out.reshape(B, S_, D_)
